# baseline (speedup 1.0000x reference)
import functools
import math

import jax
import jax.numpy as jnp
from jax import lax
from jax.experimental import pallas as pl
from jax.experimental.pallas import tpu as pltpu, tpu_sc as plsc

VOCAB_ROWS = 54
EMB_DIM = 512
SCALE = math.sqrt(float(EMB_DIM))

NC = 2
NS = 16
NW = NC * NS
LANES = 16

B_TOTAL = 1024 * 200
B_PER_W = B_TOTAL // NW
CHUNK = 32
NBUF = 4
CHUNKS_PER_W = B_PER_W // CHUNK
ITERS = CHUNKS_PER_W // NBUF
GROUPS = CHUNK // LANES

W_FLAT = VOCAB_ROWS * EMB_DIM
W_PER_W = W_FLAT // NW

_mesh = plsc.VectorSubcoreMesh(core_axis_name="c", subcore_axis_name="s")


@functools.partial(
    pl.kernel,
    mesh=_mesh,
    out_type=jax.ShapeDtypeStruct((W_FLAT,), jnp.float32),
    scratch_types=[pltpu.VMEM((W_PER_W,), jnp.float32)],
)
def _scale_table(w_hbm, out_hbm, w_v):
    wid = lax.axis_index("s") * NC + lax.axis_index("c")
    base = wid * W_PER_W
    pltpu.sync_copy(w_hbm.at[pl.ds(base, W_PER_W)], w_v)
    for j in range(W_PER_W // 16):
        w_v[pl.ds(j * 16, 16)] = w_v[pl.ds(j * 16, 16)] * SCALE
    pltpu.sync_copy(w_v, out_hbm.at[pl.ds(base, W_PER_W)])


@functools.partial(
    pl.kernel,
    mesh=_mesh,
    out_type=jax.ShapeDtypeStruct((B_TOTAL, EMB_DIM), jnp.float32),
    scratch_types=(
        [pltpu.VMEM((B_PER_W,), jnp.int32),
         pltpu.VMEM((W_FLAT,), jnp.float32)]
        + [pltpu.VMEM((CHUNK, EMB_DIM), jnp.float32) for _ in range(NBUF)]
        + [pltpu.SemaphoreType.DMA for _ in range(NBUF)]
    ),
)
def _gather(w_hbm, x_hbm, out_hbm, idx_v, table_v, *rest):
    bufs = list(rest[:NBUF])
    ssems = list(rest[NBUF:])

    wid = lax.axis_index("s") * NC + lax.axis_index("c")
    base = wid * B_PER_W
    pltpu.sync_copy(w_hbm, table_v)
    pltpu.sync_copy(x_hbm.at[pl.ds(base, B_PER_W)], idx_v)

    def fill_chunk(chunk_start, buf):
        def group_body(grp, carry):
            idx16 = idx_v[pl.ds(chunk_start + grp * LANES, LANES)]
            rowbase = idx16 * EMB_DIM
            for r in range(LANES):
                rbase = rowbase[r]
                for j in range(EMB_DIM // LANES):
                    buf[grp * LANES + r, pl.ds(j * LANES, LANES)] = (
                        table_v[pl.ds(rbase + j * LANES, LANES)]
                    )
            return carry

        lax.fori_loop(0, GROUPS, group_body, 0)

    def body(i, carry):
        c0 = i * NBUF
        for b in range(NBUF):
            @pl.when(i > 0)
            def _wait_store(b=b):
                pltpu.make_async_copy(
                    bufs[b], out_hbm.at[pl.ds(base, CHUNK)], ssems[b]
                ).wait()
            chunk_start = (c0 + b) * CHUNK
            fill_chunk(chunk_start, bufs[b])
            pltpu.make_async_copy(
                bufs[b],
                out_hbm.at[pl.ds(base + chunk_start, CHUNK)],
                ssems[b],
            ).start()
        return carry

    lax.fori_loop(0, ITERS, body, 0)
    for b in range(NBUF):
        pltpu.make_async_copy(
            bufs[b], out_hbm.at[pl.ds(base, CHUNK)], ssems[b]
        ).wait()


def kernel(x, W):
    w_scaled = _scale_table(W.reshape(-1))
    out = _gather(w_scaled, x.reshape(-1).astype(jnp.int32))
    return out.reshape(x.shape[0], x.shape[1], EMB_DIM)


# speedup vs baseline: 1.0354x; 1.0354x over previous
import functools
import math

import jax
import jax.numpy as jnp
from jax import lax
from jax.experimental import pallas as pl
from jax.experimental.pallas import tpu as pltpu, tpu_sc as plsc

VOCAB_ROWS = 54
EMB_DIM = 512
SCALE = math.sqrt(float(EMB_DIM))

NC = 2
NS = 16
NW = NC * NS
LANES = 16

B_TOTAL = 1024 * 200
B_PER_W = B_TOTAL // NW
CHUNK = 80
NBUF = 2
CHUNKS_PER_W = B_PER_W // CHUNK
ITERS = CHUNKS_PER_W // NBUF
GROUPS = CHUNK // LANES

W_FLAT = VOCAB_ROWS * EMB_DIM
W_PER_W = W_FLAT // NW

_mesh = plsc.VectorSubcoreMesh(core_axis_name="c", subcore_axis_name="s")


@functools.partial(
    pl.kernel,
    mesh=_mesh,
    out_type=jax.ShapeDtypeStruct((W_FLAT,), jnp.float32),
    scratch_types=[pltpu.VMEM((W_PER_W,), jnp.float32)],
)
def _scale_table(w_hbm, out_hbm, w_v):
    wid = lax.axis_index("s") * NC + lax.axis_index("c")
    base = wid * W_PER_W
    pltpu.sync_copy(w_hbm.at[pl.ds(base, W_PER_W)], w_v)
    for j in range(W_PER_W // 16):
        w_v[pl.ds(j * 16, 16)] = w_v[pl.ds(j * 16, 16)] * SCALE
    pltpu.sync_copy(w_v, out_hbm.at[pl.ds(base, W_PER_W)])


@functools.partial(
    pl.kernel,
    mesh=_mesh,
    out_type=jax.ShapeDtypeStruct((B_TOTAL, EMB_DIM), jnp.float32),
    scratch_types=(
        [pltpu.VMEM((B_PER_W,), jnp.int32),
         pltpu.VMEM((W_FLAT,), jnp.float32)]
        + [pltpu.VMEM((CHUNK, EMB_DIM), jnp.float32) for _ in range(NBUF)]
        + [pltpu.SemaphoreType.DMA for _ in range(NBUF)]
    ),
)
def _gather(w_hbm, x_hbm, out_hbm, idx_v, table_v, *rest):
    bufs = list(rest[:NBUF])
    ssems = list(rest[NBUF:])

    wid = lax.axis_index("s") * NC + lax.axis_index("c")
    base = wid * B_PER_W
    pltpu.sync_copy(w_hbm, table_v)
    pltpu.sync_copy(x_hbm.at[pl.ds(base, B_PER_W)], idx_v)

    def fill_chunk(chunk_start, buf):
        def group_body(grp, carry):
            idx16 = idx_v[pl.ds(chunk_start + grp * LANES, LANES)]
            rowbase = idx16 * EMB_DIM
            for r in range(LANES):
                rbase = rowbase[r]
                for j in range(EMB_DIM // LANES):
                    buf[grp * LANES + r, pl.ds(j * LANES, LANES)] = (
                        table_v[pl.ds(rbase + j * LANES, LANES)]
                    )
            return carry

        lax.fori_loop(0, GROUPS, group_body, 0)

    def body(i, carry):
        c0 = i * NBUF
        for b in range(NBUF):
            @pl.when(i > 0)
            def _wait_store(b=b):
                pltpu.make_async_copy(
                    bufs[b], out_hbm.at[pl.ds(base, CHUNK)], ssems[b]
                ).wait()
            chunk_start = (c0 + b) * CHUNK
            fill_chunk(chunk_start, bufs[b])
            pltpu.make_async_copy(
                bufs[b],
                out_hbm.at[pl.ds(base + chunk_start, CHUNK)],
                ssems[b],
            ).start()
        return carry

    lax.fori_loop(0, ITERS, body, 0)
    for b in range(NBUF):
        pltpu.make_async_copy(
            bufs[b], out_hbm.at[pl.ds(base, CHUNK)], ssems[b]
        ).wait()


def kernel(x, W):
    w_scaled = _scale_table(W.reshape(-1))
    out = _gather(w_scaled, x.reshape(-1).astype(jnp.int32))
    return out.reshape(x.shape[0], x.shape[1], EMB_DIM)


# speedup vs baseline: 2.4677x; 2.3832x over previous
import functools
import math

import jax
import jax.numpy as jnp
from jax import lax
from jax.experimental import pallas as pl
from jax.experimental.pallas import tpu as pltpu, tpu_sc as plsc

VOCAB_ROWS = 54
EMB_DIM = 512
SCALE = math.sqrt(float(EMB_DIM))

NC = 2
NS = 16
NW = NC * NS
LANES = 16

B_TOTAL = 1024 * 200
B_PER_W = B_TOTAL // NW
CHUNK = 80
NBUF = 2
CHUNKS_PER_W = B_PER_W // CHUNK
ITERS = CHUNKS_PER_W // NBUF
GROUPS = CHUNK // LANES

W_FLAT = VOCAB_ROWS * EMB_DIM
W_PER_W = W_FLAT // NW

_mesh = plsc.VectorSubcoreMesh(core_axis_name="c", subcore_axis_name="s")


@functools.partial(
    pl.kernel,
    mesh=_mesh,
    out_type=jax.ShapeDtypeStruct((W_FLAT,), jnp.float32),
    scratch_types=[pltpu.VMEM((W_PER_W,), jnp.float32)],
)
def _scale_table(w_hbm, out_hbm, w_v):
    wid = lax.axis_index("s") * NC + lax.axis_index("c")
    base = wid * W_PER_W
    pltpu.sync_copy(w_hbm.at[pl.ds(base, W_PER_W)], w_v)
    for j in range(W_PER_W // 16):
        w_v[pl.ds(j * 16, 16)] = w_v[pl.ds(j * 16, 16)] * SCALE
    pltpu.sync_copy(w_v, out_hbm.at[pl.ds(base, W_PER_W)])


@functools.partial(
    pl.kernel,
    mesh=_mesh,
    out_type=jax.ShapeDtypeStruct((B_TOTAL, EMB_DIM), jnp.float32),
    scratch_types=(
        [pltpu.VMEM((B_PER_W,), jnp.int32),
         pltpu.VMEM((W_FLAT,), jnp.float32)]
        + [pltpu.VMEM((CHUNK, EMB_DIM), jnp.float32) for _ in range(NBUF)]
        + [pltpu.SemaphoreType.DMA for _ in range(NBUF)]
    ),
)
def _gather(w_hbm, x_hbm, out_hbm, idx_v, table_v, *rest):
    bufs = list(rest[:NBUF])
    ssems = list(rest[NBUF:])

    wid = lax.axis_index("s") * NC + lax.axis_index("c")
    base = wid * B_PER_W
    pltpu.sync_copy(w_hbm, table_v)
    pltpu.sync_copy(x_hbm.at[pl.ds(base, B_PER_W)], idx_v)

    def fill_chunk(chunk_start, buf):
        def group_body(grp, carry):
            idx16 = idx_v[pl.ds(chunk_start + grp * LANES, LANES)]
            rowbase = idx16 * EMB_DIM
            for r in range(LANES):
                rbase = rowbase[r]
                vals = [
                    table_v[pl.ds(rbase + j * LANES, LANES)]
                    for j in range(EMB_DIM // LANES)
                ]
                for j in range(EMB_DIM // LANES):
                    buf[grp * LANES + r, pl.ds(j * LANES, LANES)] = vals[j]
            return carry

        lax.fori_loop(0, GROUPS, group_body, 0)

    def body(i, carry):
        c0 = i * NBUF
        for b in range(NBUF):
            @pl.when(i > 0)
            def _wait_store(b=b):
                pltpu.make_async_copy(
                    bufs[b], out_hbm.at[pl.ds(base, CHUNK)], ssems[b]
                ).wait()
            chunk_start = (c0 + b) * CHUNK
            fill_chunk(chunk_start, bufs[b])
            pltpu.make_async_copy(
                bufs[b],
                out_hbm.at[pl.ds(base + chunk_start, CHUNK)],
                ssems[b],
            ).start()
        return carry

    lax.fori_loop(0, ITERS, body, 0)
    for b in range(NBUF):
        pltpu.make_async_copy(
            bufs[b], out_hbm.at[pl.ds(base, CHUNK)], ssems[b]
        ).wait()


def kernel(x, W):
    w_scaled = _scale_table(W.reshape(-1))
    out = _gather(w_scaled, x.reshape(-1).astype(jnp.int32))
    return out.reshape(x.shape[0], x.shape[1], EMB_DIM)


# speedup vs baseline: 4.8241x; 1.9549x over previous
import functools
import math

import jax
import jax.numpy as jnp
from jax import lax
from jax.experimental import pallas as pl
from jax.experimental.pallas import tpu as pltpu, tpu_sc as plsc

VOCAB_ROWS = 54
EMB_DIM = 512
SCALE = math.sqrt(float(EMB_DIM))

NC = 2
NS = 16
NW = NC * NS
LANES = 16

B_TOTAL = 1024 * 200
B_PER_W = B_TOTAL // NW
GSZ = LANES
NSEM = 4
GROUPS_PER_W = B_PER_W // GSZ
ITERS = GROUPS_PER_W // NSEM

W_FLAT = VOCAB_ROWS * EMB_DIM
W_PER_W = W_FLAT // NW

_mesh = plsc.VectorSubcoreMesh(core_axis_name="c", subcore_axis_name="s")


@functools.partial(
    pl.kernel,
    mesh=_mesh,
    out_type=jax.ShapeDtypeStruct((W_FLAT,), jnp.float32),
    scratch_types=[pltpu.VMEM((W_PER_W,), jnp.float32)],
)
def _scale_table(w_hbm, out_hbm, w_v):
    wid = lax.axis_index("s") * NC + lax.axis_index("c")
    base = wid * W_PER_W
    pltpu.sync_copy(w_hbm.at[pl.ds(base, W_PER_W)], w_v)
    for j in range(W_PER_W // 16):
        w_v[pl.ds(j * 16, 16)] = w_v[pl.ds(j * 16, 16)] * SCALE
    pltpu.sync_copy(w_v, out_hbm.at[pl.ds(base, W_PER_W)])


@functools.partial(
    pl.kernel,
    mesh=_mesh,
    out_type=jax.ShapeDtypeStruct((B_TOTAL, EMB_DIM), jnp.float32),
    scratch_types=(
        [pltpu.VMEM((B_PER_W,), jnp.int32),
         pltpu.VMEM((VOCAB_ROWS, EMB_DIM), jnp.float32)]
        + [pltpu.SemaphoreType.DMA for _ in range(NSEM)]
    ),
)
def _gather(w_hbm, x_hbm, out_hbm, idx_v, table_v, *sems):
    wid = lax.axis_index("s") * NC + lax.axis_index("c")
    base = wid * B_PER_W
    pltpu.sync_copy(w_hbm, table_v)
    pltpu.sync_copy(x_hbm.at[pl.ds(base, B_PER_W)], idx_v)

    def drain(k):
        pltpu.make_async_copy(
            table_v.at[pl.ds(0, GSZ)], out_hbm.at[pl.ds(base, GSZ)], sems[k]
        ).wait()

    def body(i, carry):
        for k in range(NSEM):
            g = i * NSEM + k

            @pl.when(i > 0)
            def _drain(k=k):
                drain(k)

            idx16 = idx_v[pl.ds(g * GSZ, GSZ)]
            row0 = base + g * GSZ
            for r in range(GSZ):
                rid = idx16[r]
                pltpu.make_async_copy(
                    table_v.at[rid], out_hbm.at[row0 + r], sems[k]
                ).start()
        return carry

    lax.fori_loop(0, ITERS, body, 0)
    for k in range(NSEM):
        drain(k)


def kernel(x, W):
    w_scaled = _scale_table(W.reshape(-1)).reshape(VOCAB_ROWS, EMB_DIM)
    out = _gather(w_scaled, x.reshape(-1).astype(jnp.int32))
    return out.reshape(x.shape[0], x.shape[1], EMB_DIM)
